# baseline (speedup 1.0000x reference)
import functools

import jax
import jax.numpy as jnp
from jax import lax
from jax.experimental import pallas as pl
from jax.experimental.pallas import tpu as pltpu
from jax.experimental.pallas import tpu_sc as plsc

D_MODEL = 64
SEQ = 200
BATCH = 4096
LANES = 16
NUM_CORES = 2
NUM_SUBCORES = 16
NW = NUM_CORES * NUM_SUBCORES
TOTAL_ROWS = BATCH * SEQ
ROWS_PER_W = TOTAL_ROWS // NW
CHUNK = 128
N_CHUNKS = ROWS_PER_W // CHUNK


def _positional_encoding(length, depth):
    half = depth // 2
    positions = jnp.arange(length, dtype=jnp.float32)[:, None]
    depths = jnp.arange(half, dtype=jnp.float32)[None, :]
    angle_rates = 1.0 / (10000.0 ** depths)
    angle_rads = positions * angle_rates
    return jnp.concatenate([jnp.sin(angle_rads), jnp.cos(angle_rads)], axis=-1)


def _sc_body(table_hbm, idx_hbm, pe_hbm, out_hbm, idx_v, pe_v, rows_v, gsem):
    wid = lax.axis_index("s") * NUM_CORES + lax.axis_index("c")
    pltpu.sync_copy(idx_hbm.at[wid], idx_v)
    pltpu.sync_copy(pe_hbm, pe_v)
    out_base = wid * ROWS_PER_W

    @pl.loop(0, N_CHUNKS)
    def _chunk(g):
        pltpu.async_copy(table_hbm.at[idx_v.at[g]], rows_v, gsem).wait()
        p0 = lax.rem(g * CHUNK, SEQ)

        @pl.loop(0, CHUNK)
        def _row(r):
            pr = p0 + r
            pr = jnp.where(pr >= SEQ, pr - SEQ, pr)
            for c in range(0, D_MODEL, LANES):
                rows_v[r, pl.ds(c, LANES)] = (
                    rows_v[r, pl.ds(c, LANES)] * 8.0 + pe_v[pr, pl.ds(c, LANES)]
                )

        pltpu.sync_copy(rows_v, out_hbm.at[pl.ds(out_base + g * CHUNK, CHUNK)])


def kernel(x, table):
    idx = x.reshape(NW, N_CHUNKS, CHUNK)
    pe = _positional_encoding(SEQ, D_MODEL)

    mesh = plsc.VectorSubcoreMesh(
        core_axis_name="c", subcore_axis_name="s",
        num_cores=NUM_CORES, num_subcores=NUM_SUBCORES,
    )
    k = pl.kernel(
        _sc_body,
        out_type=jax.ShapeDtypeStruct((TOTAL_ROWS, D_MODEL), jnp.float32),
        mesh=mesh,
        scratch_types=[
            pltpu.VMEM((N_CHUNKS, CHUNK), jnp.int32),
            pltpu.VMEM((SEQ, D_MODEL), jnp.float32),
            pltpu.VMEM((CHUNK, D_MODEL), jnp.float32),
            pltpu.SemaphoreType.DMA,
        ],
        compiler_params=pltpu.CompilerParams(use_tc_tiling_on_sc=False),
    )
    out = k(table, idx, pe)
    return out.reshape(BATCH, SEQ, D_MODEL)


# speedup vs baseline: 1.1851x; 1.1851x over previous
import functools

import jax
import jax.numpy as jnp
from jax import lax
from jax.experimental import pallas as pl
from jax.experimental.pallas import tpu as pltpu
from jax.experimental.pallas import tpu_sc as plsc

D_MODEL = 64
SEQ = 200
BATCH = 4096
LANES = 16
NUM_CORES = 2
NUM_SUBCORES = 16
NW = NUM_CORES * NUM_SUBCORES
TOTAL_ROWS = BATCH * SEQ
ROWS_PER_W = TOTAL_ROWS // NW
CHUNK = 128
N_CHUNKS = ROWS_PER_W // CHUNK


def _positional_encoding(length, depth):
    half = depth // 2
    positions = jnp.arange(length, dtype=jnp.float32)[:, None]
    depths = jnp.arange(half, dtype=jnp.float32)[None, :]
    angle_rates = 1.0 / (10000.0 ** depths)
    angle_rads = positions * angle_rates
    return jnp.concatenate([jnp.sin(angle_rads), jnp.cos(angle_rads)], axis=-1)


NBUF = 4


def _sc_body(table_hbm, idx_hbm, pe_hbm, out_hbm,
             idx_v, pe_v, bufs, gsems, ssems):
    wid = lax.axis_index("s") * NUM_CORES + lax.axis_index("c")
    pltpu.sync_copy(idx_hbm.at[wid], idx_v)
    pltpu.sync_copy(pe_hbm, pe_v)
    out_base = wid * ROWS_PER_W

    def start_gather(g, p):
        pltpu.async_copy(table_hbm.at[idx_v.at[g]], bufs[p], gsems[p])

    def wait_gather(p):
        pltpu.make_async_copy(table_hbm.at[idx_v.at[0]], bufs[p], gsems[p]).wait()

    def start_store(g, p):
        pltpu.async_copy(bufs[p], out_hbm.at[pl.ds(out_base + g * CHUNK, CHUNK)],
                         ssems[p])

    def wait_store(p):
        pltpu.make_async_copy(bufs[p], out_hbm.at[pl.ds(out_base, CHUNK)],
                              ssems[p]).wait()

    def compute(g, p):
        buf = bufs[p]
        p0 = lax.rem(g * CHUNK, SEQ)

        @pl.loop(0, CHUNK, unroll=4)
        def _row(r):
            pr = p0 + r
            pr = jnp.where(pr >= SEQ, pr - SEQ, pr)
            for c in range(0, D_MODEL, LANES):
                buf[r, pl.ds(c, LANES)] = (
                    buf[r, pl.ds(c, LANES)] * 8.0 + pe_v[pr, pl.ds(c, LANES)]
                )

    start_gather(0, 0)

    @pl.loop(0, N_CHUNKS // NBUF)
    def _ring(h):
        for p in range(NBUF):
            g = h * NBUF + p
            wait_gather(p)
            nxt = (p + 1) % NBUF
            @pl.when(g >= NBUF - 1)
            def _():
                wait_store(nxt)

            @pl.when(g + 1 < N_CHUNKS)
            def _():
                start_gather(g + 1, nxt)

            compute(g, p)
            start_store(g, p)

    for p in range(1, NBUF):
        wait_store(p)


def kernel(x, table):
    idx = x.reshape(NW, N_CHUNKS, CHUNK)
    pe = _positional_encoding(SEQ, D_MODEL)

    mesh = plsc.VectorSubcoreMesh(
        core_axis_name="c", subcore_axis_name="s",
        num_cores=NUM_CORES, num_subcores=NUM_SUBCORES,
    )
    k = pl.kernel(
        _sc_body,
        out_type=jax.ShapeDtypeStruct((TOTAL_ROWS, D_MODEL), jnp.float32),
        mesh=mesh,
        scratch_types=[
            pltpu.VMEM((N_CHUNKS, CHUNK), jnp.int32),
            pltpu.VMEM((SEQ, D_MODEL), jnp.float32),
            [pltpu.VMEM((CHUNK, D_MODEL), jnp.float32) for _ in range(NBUF)],
            [pltpu.SemaphoreType.DMA for _ in range(NBUF)],
            [pltpu.SemaphoreType.DMA for _ in range(NBUF)],
        ],
        compiler_params=pltpu.CompilerParams(use_tc_tiling_on_sc=False),
    )
    out = k(table, idx, pe)
    return out.reshape(BATCH, SEQ, D_MODEL)
